# Initial kernel scaffold; baseline (speedup 1.0000x reference)
import functools

import jax
import jax.numpy as jnp
from jax import lax
from jax.experimental import pallas as pl
from jax.experimental.pallas import tpu as pltpu
from jax.experimental.pallas import tpu_sc as plsc

_N = 10000
_NC, _NS = 2, 16
_NW = _NC * _NS
_CH = 128
_RPT = 626
_NPAD = _NS * _RPT
_BN = 1000
_G = _N // _BN
_EPS = 1e-5


@functools.lru_cache(maxsize=None)
def _make_propagate(d: int, t: int):
    mesh = plsc.VectorSubcoreMesh(core_axis_name="c", subcore_axis_name="s")

    @functools.partial(
        pl.kernel,
        out_type=jax.ShapeDtypeStruct((_NC, _NPAD, d), jnp.float32),
        mesh=mesh,
        scratch_types=[
            pltpu.VMEM((t, _CH), jnp.int32),
            pltpu.VMEM((t, _CH), jnp.int32),
            pltpu.VMEM((2, _CH, d), jnp.float32),
            pltpu.VMEM_SHARED((_NPAD, d), jnp.float32),
            pltpu.SemaphoreType.DMA,
            pltpu.SemaphoreType.DMA,
        ],
    )
    def prop(table_hbm, src_hbm, dst_hbm, zrows_hbm, out_hbm,
             src_v, dst_v, rows_v, acc_sh, sem0, sem1):
        c = lax.axis_index("c")
        s = lax.axis_index("s")
        wid = c * _NS + s

        pltpu.sync_copy(src_hbm.at[pl.ds(wid * t, t)], src_v)
        pltpu.sync_copy(dst_hbm.at[pl.ds(wid * t, t)], dst_v)
        pltpu.sync_copy(zrows_hbm, acc_sh.at[pl.ds(s * _RPT, _RPT)])
        plsc.subcore_barrier()

        sems = (sem0, sem1)
        pltpu.async_copy(table_hbm.at[src_v.at[0]], rows_v.at[0], sem0)
        for j in range(t):
            b = j % 2
            if j + 1 < t:
                nb = (j + 1) % 2
                pltpu.async_copy(table_hbm.at[src_v.at[j + 1]],
                                 rows_v.at[nb], sems[nb])
            pltpu.make_async_copy(table_hbm.at[src_v.at[j]],
                                  rows_v.at[b], sems[b]).wait()
            pltpu.sync_copy(rows_v.at[b], acc_sh.at[dst_v.at[j]], add=True)
        plsc.subcore_barrier()

        pltpu.sync_copy(acc_sh.at[pl.ds(s * _RPT, _RPT)],
                        out_hbm.at[c, pl.ds(s * _RPT, _RPT)])

    return prop


def _dinv_block(deg_ref):
    deg = deg_ref[0, :, 0:1] + deg_ref[1, :, 0:1] + 1.0
    return lax.rsqrt(deg)


def _pre_body(x_ref, w_ref, deg_ref, o_ref):
    h = jnp.dot(x_ref[...], w_ref[...], preferred_element_type=jnp.float32)
    o_ref[...] = h * _dinv_block(deg_ref)


def _mid_body(raw_ref, st_ref, g_ref, be_ref, w_ref, deg_ref, o_ref):
    mu = st_ref[0:1, :]
    rstd = st_ref[1:2, :]
    h = (raw_ref[...] - mu) * rstd * g_ref[...] + be_ref[...]
    h = jnp.maximum(h, 0.0)
    o = jnp.dot(h, w_ref[...], preferred_element_type=jnp.float32)
    o_ref[...] = o * _dinv_block(deg_ref)


def _stats_body(acc_ref, hs_ref, deg_ref, b_ref, raw_ref, st_ref, sums):
    i = pl.program_id(0)
    raw = (acc_ref[0] + acc_ref[1] + hs_ref[...]) * _dinv_block(deg_ref) \
        + b_ref[...]
    raw_ref[...] = raw
    part = jnp.stack([jnp.sum(raw, axis=0), jnp.sum(raw * raw, axis=0)])

    @pl.when(i == 0)
    def _():
        sums[...] = jnp.zeros_like(sums)

    sums[...] += part

    @pl.when(i == _G - 1)
    def _():
        mu = sums[0:1, :] / _N
        var = sums[1:2, :] / _N - mu * mu
        st_ref[...] = jnp.concatenate([mu, lax.rsqrt(var + _EPS)], axis=0)


def _final_body(acc_ref, hs_ref, deg_ref, b_ref, o_ref):
    raw = (acc_ref[0] + acc_ref[1] + hs_ref[...]) * _dinv_block(deg_ref) \
        + b_ref[...]
    o_ref[...] = raw[:, 0:2]


def _row_spec(d):
    return pl.BlockSpec((_BN, d), lambda i: (i, 0))


def _full_spec(shape):
    nd = len(shape)
    return pl.BlockSpec(shape, lambda i, _n=nd: (0,) * _n)


def _deg_spec():
    return pl.BlockSpec((2, _BN, 16), lambda i: (0, i, 0))


def _matmul_pre(x, w, degp):
    d_in, d_out = w.shape
    return pl.pallas_call(
        _pre_body,
        grid=(_G,),
        in_specs=[_row_spec(d_in), _full_spec((d_in, d_out)), _deg_spec()],
        out_specs=_row_spec(d_out),
        out_shape=jax.ShapeDtypeStruct((_N, d_out), jnp.float32),
    )(x, w, degp)


def _matmul_mid(raw, stats, g, be, w, degp):
    d_in, d_out = w.shape
    return pl.pallas_call(
        _mid_body,
        grid=(_G,),
        in_specs=[_row_spec(d_in), _full_spec((2, d_in)),
                  _full_spec((1, d_in)), _full_spec((1, d_in)),
                  _full_spec((d_in, d_out)), _deg_spec()],
        out_specs=_row_spec(d_out),
        out_shape=jax.ShapeDtypeStruct((_N, d_out), jnp.float32),
    )(raw, stats, g, be, w, degp)


def _stats(accp, hs, degp, b):
    d = hs.shape[1]
    return pl.pallas_call(
        _stats_body,
        grid=(_G,),
        in_specs=[pl.BlockSpec((2, _BN, d), lambda i: (0, i, 0)),
                  _row_spec(d), _deg_spec(), _full_spec((1, d))],
        out_specs=[_row_spec(d), _full_spec((2, d))],
        out_shape=[jax.ShapeDtypeStruct((_N, d), jnp.float32),
                   jax.ShapeDtypeStruct((2, d), jnp.float32)],
        scratch_shapes=[pltpu.VMEM((2, d), jnp.float32)],
    )(accp, hs, degp, b)


def _final(accp, hs, degp, b):
    d = hs.shape[1]
    return pl.pallas_call(
        _final_body,
        grid=(_G,),
        in_specs=[pl.BlockSpec((2, _BN, d), lambda i: (0, i, 0)),
                  _row_spec(d), _deg_spec(), _full_spec((1, d))],
        out_specs=pl.BlockSpec((_BN, 2), lambda i: (i, 0)),
        out_shape=jax.ShapeDtypeStruct((_N, 2), jnp.float32),
    )(accp, hs, degp, b)


def kernel(x, edge_index, params):
    e = edge_index.shape[1]
    t = -(-e // (_NW * _CH))
    epad = _NW * _CH * t
    src = jnp.concatenate(
        [edge_index[0], jnp.zeros((epad - e,), jnp.int32)]).reshape(
            _NW * t, _CH)
    dst = jnp.concatenate(
        [edge_index[1], jnp.full((epad - e,), _N, jnp.int32)]).reshape(
            _NW * t, _CH)

    prop128 = _make_propagate(128, t)
    prop64 = _make_propagate(64, t)
    prop16 = _make_propagate(16, t)
    z128 = jnp.zeros((_RPT, 128), jnp.float32)
    z64 = jnp.zeros((_RPT, 64), jnp.float32)
    z16 = jnp.zeros((_RPT, 16), jnp.float32)

    degp = prop16(jnp.ones((_N, 16), jnp.float32), src, dst, z16)

    w5 = jnp.concatenate(
        [params['W5'], jnp.zeros((params['W5'].shape[0], 14), jnp.float32)],
        axis=1)
    b5 = jnp.concatenate([params['b5'], jnp.zeros((14,), jnp.float32)])

    ws = [params['W1'], params['W2'], params['W3'], params['W4'], w5]
    bs = [params['b1'], params['b2'], params['b3'], params['b4'], b5]
    props = [prop128, prop128, prop128, prop64, prop16]
    zs = [z128, z128, z128, z64, z16]

    hs = _matmul_pre(x, ws[0], degp)
    for i in range(5):
        accp = props[i](hs, src, dst, zs[i])
        bvec = bs[i].reshape(1, -1)
        if i < 4:
            raw, stats = _stats(accp, hs, degp, bvec)
            hs = _matmul_mid(raw, stats, params[f'g{i + 1}'].reshape(1, -1),
                             params[f'be{i + 1}'].reshape(1, -1),
                             ws[i + 1], degp)
        else:
            out = _final(accp, hs, degp, bvec)
    return out


# speedup vs baseline: 17.4502x; 17.4502x over previous
import functools

import jax
import jax.numpy as jnp
from jax import lax
from jax.experimental import pallas as pl
from jax.experimental.pallas import tpu as pltpu
from jax.experimental.pallas import tpu_sc as plsc

_N = 10000
_NC, _NS = 2, 16
_NW = _NC * _NS
_CH = 128
_RPT = 632
_NPAD = _NS * _RPT
_BN = 1000
_G = _N // _BN
_EPS = 1e-5
_P = 40


@functools.lru_cache(maxsize=None)
def _make_propagate(d: int, t: int):
    mesh = plsc.VectorSubcoreMesh(core_axis_name="c", subcore_axis_name="s")

    @functools.partial(
        pl.kernel,
        out_type=jax.ShapeDtypeStruct((_NC, _NPAD, d), jnp.float32),
        mesh=mesh,
        scratch_types=[
            pltpu.VMEM((t // 2, _CH), jnp.int32),
            pltpu.VMEM((t // 2, _CH), jnp.int32),
            pltpu.VMEM((2, _CH, d), jnp.float32),
            pltpu.VMEM_SHARED((_NPAD, d), jnp.float32),
            pltpu.SemaphoreType.DMA,
            pltpu.SemaphoreType.DMA,
            pltpu.SemaphoreType.DMA,
            pltpu.SemaphoreType.DMA,
        ],
        compiler_params=pltpu.CompilerParams(use_tc_tiling_on_sc=False),
    )
    def prop(table_hbm, src_hbm, dst_hbm, zrows_hbm, out_hbm,
             src_v, dst_v, rows_v, acc_sh, gsem0, gsem1, ssem0, ssem1):
        c = lax.axis_index("c")
        s = lax.axis_index("s")
        wid = c * _NS + s
        rs = pl.ds(s * _RPT, _RPT)

        pltpu.sync_copy(zrows_hbm, acc_sh.at[rs])
        plsc.subcore_barrier()

        gsems = (gsem0, gsem1)
        ssems = (ssem0, ssem1)
        half = t // 2
        for phase in range(2):
            base = wid * t + phase * half
            pltpu.sync_copy(src_hbm.at[pl.ds(base, half)], src_v)
            pltpu.sync_copy(dst_hbm.at[pl.ds(base, half)], dst_v)
            pltpu.async_copy(table_hbm.at[src_v.at[0]], rows_v.at[0], gsem0)
            scat = [None, None]
            for j in range(half):
                b = j % 2
                nb = (j + 1) % 2
                if j + 1 < half:
                    if scat[nb] is not None:
                        scat[nb].wait()
                        scat[nb] = None
                    pltpu.async_copy(table_hbm.at[src_v.at[j + 1]],
                                     rows_v.at[nb], gsems[nb])
                pltpu.make_async_copy(table_hbm.at[src_v.at[j]],
                                      rows_v.at[b], gsems[b]).wait()
                scat[b] = pltpu.async_copy(
                    rows_v.at[b], acc_sh.at[dst_v.at[j]], ssems[b],
                    add=True)
            for b in range(2):
                if scat[b] is not None:
                    scat[b].wait()
        plsc.subcore_barrier()

        pltpu.sync_copy(acc_sh.at[rs], out_hbm.at[c, rs])

    return prop


@functools.lru_cache(maxsize=None)
def _make_propagate_cols(dcol: int, t2: int):
    mesh = plsc.VectorSubcoreMesh(core_axis_name="c", subcore_axis_name="s")

    @functools.partial(
        pl.kernel,
        out_type=jax.ShapeDtypeStruct((_NC, _NPAD, dcol), jnp.float32),
        mesh=mesh,
        scratch_types=[
            pltpu.VMEM((_P, _CH), jnp.int32),
            pltpu.VMEM((_P, _CH), jnp.int32),
            pltpu.VMEM((2, _CH, dcol), jnp.float32),
            pltpu.VMEM_SHARED((_NPAD, dcol), jnp.float32),
            pltpu.VMEM_SHARED((_NPAD, dcol), jnp.float32),
            pltpu.SemaphoreType.DMA,
            pltpu.SemaphoreType.DMA,
            pltpu.SemaphoreType.DMA,
            pltpu.SemaphoreType.DMA,
        ],
        compiler_params=pltpu.CompilerParams(use_tc_tiling_on_sc=False),
    )
    def prop(table_hbm, src_hbm, dst_hbm, zrows_hbm, out_hbm,
             src_v, dst_v, rows_v, tbl_sh, acc_sh,
             gsem0, gsem1, ssem0, ssem1):
        c = lax.axis_index("c")
        s = lax.axis_index("s")
        rs = pl.ds(s * _RPT, _RPT)

        pltpu.sync_copy(table_hbm.at[rs, pl.ds(c * dcol, dcol)],
                        tbl_sh.at[rs])
        pltpu.sync_copy(zrows_hbm, acc_sh.at[rs])
        plsc.subcore_barrier()

        gsems = (gsem0, gsem1)
        ssems = (ssem0, ssem1)
        for piece in range(t2 // _P):
            base = s * t2 + piece * _P
            pltpu.sync_copy(src_hbm.at[pl.ds(base, _P)], src_v)
            pltpu.sync_copy(dst_hbm.at[pl.ds(base, _P)], dst_v)
            pltpu.async_copy(tbl_sh.at[src_v.at[0]], rows_v.at[0], gsem0)
            scat = [None, None]
            for j in range(_P):
                b = j % 2
                nb = (j + 1) % 2
                if j + 1 < _P:
                    if scat[nb] is not None:
                        scat[nb].wait()
                        scat[nb] = None
                    pltpu.async_copy(tbl_sh.at[src_v.at[j + 1]],
                                     rows_v.at[nb], gsems[nb])
                pltpu.make_async_copy(tbl_sh.at[src_v.at[j]],
                                      rows_v.at[b], gsems[b]).wait()
                scat[b] = pltpu.async_copy(
                    rows_v.at[b], acc_sh.at[dst_v.at[j]], ssems[b],
                    add=True)
            for b in range(2):
                if scat[b] is not None:
                    scat[b].wait()
        plsc.subcore_barrier()

        pltpu.sync_copy(acc_sh.at[rs], out_hbm.at[c, rs])

    return prop


def _dinv_block(deg_ref):
    deg = deg_ref[0, :, 0:1] + deg_ref[1, :, 0:1] + 1.0
    return lax.rsqrt(deg)


def _plain_body(x_ref, w_ref, o_ref):
    o_ref[...] = jnp.dot(x_ref[...], w_ref[...],
                         preferred_element_type=jnp.float32)


def _scale_body(z_ref, deg_ref, o_ref):
    o_ref[...] = z_ref[...] * _dinv_block(deg_ref)


def _mid_body(raw_ref, st_ref, g_ref, be_ref, w_ref, deg_ref, o_ref):
    mu = st_ref[0:1, :]
    rstd = st_ref[1:2, :]
    h = (raw_ref[...] - mu) * rstd * g_ref[...] + be_ref[...]
    h = jnp.maximum(h, 0.0)
    o = jnp.dot(h, w_ref[...], preferred_element_type=jnp.float32)
    o_ref[...] = o * _dinv_block(deg_ref)


def _stats_body(acc_ref, hs_ref, deg_ref, b_ref, raw_ref, st_ref, sums):
    i = pl.program_id(0)
    acc = jnp.concatenate([acc_ref[0], acc_ref[1]], axis=1)
    raw = (acc + hs_ref[...]) * _dinv_block(deg_ref) + b_ref[...]
    raw_ref[...] = raw
    part = jnp.stack([jnp.sum(raw, axis=0), jnp.sum(raw * raw, axis=0)])

    @pl.when(i == 0)
    def _():
        sums[...] = jnp.zeros_like(sums)

    sums[...] += part

    @pl.when(i == _G - 1)
    def _():
        mu = sums[0:1, :] / _N
        var = sums[1:2, :] / _N - mu * mu
        st_ref[...] = jnp.concatenate([mu, lax.rsqrt(var + _EPS)], axis=0)


def _final_body(acc_ref, hs_ref, deg_ref, b_ref, o_ref):
    acc = jnp.concatenate([acc_ref[0], acc_ref[1]], axis=1)
    raw = (acc + hs_ref[...]) * _dinv_block(deg_ref) + b_ref[...]
    o_ref[...] = raw[:, 0:2]


def _row_spec(d):
    return pl.BlockSpec((_BN, d), lambda i: (i, 0))


def _full_spec(shape):
    nd = len(shape)
    return pl.BlockSpec(shape, lambda i, _n=nd: (0,) * _n)


def _deg_spec():
    return pl.BlockSpec((2, _BN, 8), lambda i: (0, i, 0))


def _matmul_plain(x, w):
    d_in, d_out = w.shape
    return pl.pallas_call(
        _plain_body,
        grid=(_G,),
        in_specs=[_row_spec(d_in), _full_spec((d_in, d_out))],
        out_specs=_row_spec(d_out),
        out_shape=jax.ShapeDtypeStruct((_NPAD, d_out), jnp.float32),
    )(x, w)


def _scale(z, degp):
    d = z.shape[1]
    return pl.pallas_call(
        _scale_body,
        grid=(_G,),
        in_specs=[_row_spec(d), _deg_spec()],
        out_specs=_row_spec(d),
        out_shape=jax.ShapeDtypeStruct((_NPAD, d), jnp.float32),
    )(z, degp)


def _matmul_mid(raw, stats, g, be, w, degp):
    d_in, d_out = w.shape
    return pl.pallas_call(
        _mid_body,
        grid=(_G,),
        in_specs=[_row_spec(d_in), _full_spec((2, d_in)),
                  _full_spec((1, d_in)), _full_spec((1, d_in)),
                  _full_spec((d_in, d_out)), _deg_spec()],
        out_specs=_row_spec(d_out),
        out_shape=jax.ShapeDtypeStruct((_NPAD, d_out), jnp.float32),
    )(raw, stats, g, be, w, degp)


def _stats(accp, hs, degp, b):
    d = hs.shape[1]
    dcol = accp.shape[2]
    return pl.pallas_call(
        _stats_body,
        grid=(_G,),
        in_specs=[pl.BlockSpec((2, _BN, dcol), lambda i: (0, i, 0)),
                  _row_spec(d), _deg_spec(), _full_spec((1, d))],
        out_specs=[_row_spec(d), _full_spec((2, d))],
        out_shape=[jax.ShapeDtypeStruct((_NPAD, d), jnp.float32),
                   jax.ShapeDtypeStruct((2, d), jnp.float32)],
        scratch_shapes=[pltpu.VMEM((2, d), jnp.float32)],
    )(accp, hs, degp, b)


def _final(accp, hs, degp, b):
    d = hs.shape[1]
    dcol = accp.shape[2]
    return pl.pallas_call(
        _final_body,
        grid=(_G,),
        in_specs=[pl.BlockSpec((2, _BN, dcol), lambda i: (0, i, 0)),
                  _row_spec(d), _deg_spec(), _full_spec((1, d))],
        out_specs=pl.BlockSpec((_BN, 2), lambda i: (i, 0)),
        out_shape=jax.ShapeDtypeStruct((_N, 2), jnp.float32),
    )(accp, hs, degp, b)


def kernel(x, edge_index, params):
    e = edge_index.shape[1]
    t = -(-e // (_NW * _CH))
    t = (t + 7) // 8 * 8
    epad = _NW * _CH * t
    t2 = 2 * t
    src = jnp.concatenate(
        [edge_index[0], jnp.zeros((epad - e,), jnp.int32)]).reshape(
            _NW * t, _CH)
    dst = jnp.concatenate(
        [edge_index[1], jnp.full((epad - e,), _N, jnp.int32)]).reshape(
            _NW * t, _CH)

    prop8 = _make_propagate(8, t)
    prop_c64 = _make_propagate_cols(64, t2)
    prop_c32 = _make_propagate_cols(32, t2)
    z64 = jnp.zeros((_RPT, 64), jnp.float32)
    z32 = jnp.zeros((_RPT, 32), jnp.float32)
    z8 = jnp.zeros((_RPT, 8), jnp.float32)

    xpad = jnp.concatenate(
        [x, jnp.zeros((_NPAD - _N, x.shape[1]), jnp.float32)])

    w5 = jnp.concatenate(
        [params['W5'], jnp.zeros((params['W5'].shape[0], 62), jnp.float32)],
        axis=1)
    b5 = jnp.concatenate([params['b5'], jnp.zeros((62,), jnp.float32)])

    ws = [params['W1'], params['W2'], params['W3'], params['W4'], w5]
    bs = [params['b1'], params['b2'], params['b3'], params['b4'], b5]
    props = [prop_c64, prop_c64, prop_c64, prop_c32, prop_c32]
    zs = [z64, z64, z64, z32, z32]

    degp = prop8(jnp.ones((_N, 8), jnp.float32), src, dst, z8)
    z1 = _matmul_plain(xpad, ws[0])
    hs = _scale(z1, degp)
    for i in range(5):
        accp = props[i](hs, src, dst, zs[i])
        bvec = bs[i].reshape(1, -1)
        if i < 4:
            raw, stats = _stats(accp, hs, degp, bvec)
            hs = _matmul_mid(raw, stats, params[f'g{i + 1}'].reshape(1, -1),
                             params[f'be{i + 1}'].reshape(1, -1),
                             ws[i + 1], degp)
        else:
            out = _final(accp, hs, degp, bvec)
    return out


# speedup vs baseline: 18.3662x; 1.0525x over previous
import functools

import jax
import jax.numpy as jnp
from jax import lax
from jax.experimental import pallas as pl
from jax.experimental.pallas import tpu as pltpu
from jax.experimental.pallas import tpu_sc as plsc

_N = 10000
_NC, _NS = 2, 16
_NW = _NC * _NS
_CH = 128
_RPT = 632
_NPAD = _NS * _RPT
_BN = 1000
_G = _N // _BN
_EPS = 1e-5
_P = 40


@functools.lru_cache(maxsize=None)
def _make_degree(d: int, t: int):
    mesh = plsc.VectorSubcoreMesh(core_axis_name="c", subcore_axis_name="s")

    @functools.partial(
        pl.kernel,
        out_type=jax.ShapeDtypeStruct((_NC, _NPAD, d), jnp.float32),
        mesh=mesh,
        scratch_types=[
            pltpu.VMEM((t, _CH), jnp.int32),
            pltpu.VMEM((_CH, d), jnp.float32),
            pltpu.VMEM_SHARED((_NPAD, d), jnp.float32),
            pltpu.SemaphoreType.DMA,
            pltpu.SemaphoreType.DMA,
        ],
        compiler_params=pltpu.CompilerParams(use_tc_tiling_on_sc=False),
    )
    def deg(ones_hbm, dst_hbm, zrows_hbm, out_hbm,
            dst_v, ones_v, acc_sh, ssem0, ssem1):
        c = lax.axis_index("c")
        s = lax.axis_index("s")
        wid = c * _NS + s
        rs = pl.ds(s * _RPT, _RPT)

        pltpu.sync_copy(ones_hbm, ones_v)
        pltpu.sync_copy(dst_hbm.at[pl.ds(wid * t, t)], dst_v)
        pltpu.sync_copy(zrows_hbm, acc_sh.at[rs])
        plsc.subcore_barrier()

        ssems = (ssem0, ssem1)
        scat = [None, None]
        for j in range(t):
            b = j % 2
            if scat[b] is not None:
                scat[b].wait()
            scat[b] = pltpu.async_copy(
                ones_v, acc_sh.at[dst_v.at[j]], ssems[b], add=True)
        for b in range(2):
            if scat[b] is not None:
                scat[b].wait()
        plsc.subcore_barrier()

        pltpu.sync_copy(acc_sh.at[rs], out_hbm.at[c, rs])

    return deg


@functools.lru_cache(maxsize=None)
def _make_propagate_cols(dcol: int, t2: int):
    mesh = plsc.VectorSubcoreMesh(core_axis_name="c", subcore_axis_name="s")

    @functools.partial(
        pl.kernel,
        out_type=jax.ShapeDtypeStruct((_NC, _NPAD, dcol), jnp.float32),
        mesh=mesh,
        scratch_types=[
            pltpu.VMEM((_P, _CH), jnp.int32),
            pltpu.VMEM((_P, _CH), jnp.int32),
            pltpu.VMEM((2, _CH, dcol), jnp.float32),
            pltpu.VMEM_SHARED((_NPAD, dcol), jnp.float32),
            pltpu.VMEM_SHARED((_NPAD, dcol), jnp.float32),
            pltpu.SemaphoreType.DMA,
            pltpu.SemaphoreType.DMA,
            pltpu.SemaphoreType.DMA,
            pltpu.SemaphoreType.DMA,
        ],
        compiler_params=pltpu.CompilerParams(use_tc_tiling_on_sc=False),
    )
    def prop(table_hbm, src_hbm, dst_hbm, zrows_hbm, out_hbm,
             src_v, dst_v, rows_v, tbl_sh, acc_sh,
             gsem0, gsem1, ssem0, ssem1):
        c = lax.axis_index("c")
        s = lax.axis_index("s")
        rs = pl.ds(s * _RPT, _RPT)

        pltpu.sync_copy(table_hbm.at[rs, pl.ds(c * dcol, dcol)],
                        tbl_sh.at[rs])
        pltpu.sync_copy(zrows_hbm, acc_sh.at[rs])
        plsc.subcore_barrier()

        gsems = (gsem0, gsem1)
        ssems = (ssem0, ssem1)
        for piece in range(t2 // _P):
            base = s * t2 + piece * _P
            pltpu.sync_copy(src_hbm.at[pl.ds(base, _P)], src_v)
            pltpu.sync_copy(dst_hbm.at[pl.ds(base, _P)], dst_v)
            pltpu.async_copy(tbl_sh.at[src_v.at[0]], rows_v.at[0], gsem0)
            scat = [None, None]
            for j in range(_P):
                b = j % 2
                nb = (j + 1) % 2
                if j + 1 < _P:
                    if scat[nb] is not None:
                        scat[nb].wait()
                        scat[nb] = None
                    pltpu.async_copy(tbl_sh.at[src_v.at[j + 1]],
                                     rows_v.at[nb], gsems[nb])
                pltpu.make_async_copy(tbl_sh.at[src_v.at[j]],
                                      rows_v.at[b], gsems[b]).wait()
                scat[b] = pltpu.async_copy(
                    rows_v.at[b], acc_sh.at[dst_v.at[j]], ssems[b],
                    add=True)
            for b in range(2):
                if scat[b] is not None:
                    scat[b].wait()
        plsc.subcore_barrier()

        pltpu.sync_copy(acc_sh.at[rs], out_hbm.at[c, rs])

    return prop


def _dinv_block(deg_ref):
    deg = deg_ref[0, :, 0:1] + deg_ref[1, :, 0:1] + 1.0
    return lax.rsqrt(deg)


def _plain_body(x_ref, w_ref, o_ref):
    o_ref[...] = jnp.dot(x_ref[...], w_ref[...],
                         preferred_element_type=jnp.float32)


def _scale_body(z_ref, deg_ref, o_ref):
    o_ref[...] = z_ref[...] * _dinv_block(deg_ref)


def _fused_body(acc_ref, hs_ref, deg_ref, b_ref, g_ref, be_ref, w_ref,
                o_ref, rawbuf, sums, stats):
    i = pl.program_id(0)

    @pl.when(i < _G)
    def _():
        acc = jnp.concatenate([acc_ref[0], acc_ref[1]], axis=1)
        raw = (acc + hs_ref[...]) * _dinv_block(deg_ref) + b_ref[...]
        rawbuf[pl.ds(i * _BN, _BN), :] = raw
        part = jnp.stack([jnp.sum(raw, axis=0), jnp.sum(raw * raw, axis=0)])

        @pl.when(i == 0)
        def _():
            sums[...] = jnp.zeros_like(sums)

        sums[...] += part

        @pl.when(i == _G - 1)
        def _():
            mu = sums[0:1, :] / _N
            var = sums[1:2, :] / _N - mu * mu
            stats[...] = jnp.concatenate([mu, lax.rsqrt(var + _EPS)],
                                         axis=0)

    @pl.when(i >= _G)
    def _():
        raw = rawbuf[pl.ds((i - _G) * _BN, _BN), :]
        h = (raw - stats[0:1, :]) * stats[1:2, :] * g_ref[...] + be_ref[...]
        h = jnp.maximum(h, 0.0)
        o = jnp.dot(h, w_ref[...], preferred_element_type=jnp.float32)
        o_ref[...] = o * _dinv_block(deg_ref)


def _final_body(acc_ref, hs_ref, deg_ref, b_ref, o_ref):
    acc = jnp.concatenate([acc_ref[0], acc_ref[1]], axis=1)
    raw = (acc + hs_ref[...]) * _dinv_block(deg_ref) + b_ref[...]
    o_ref[...] = raw[:, 0:2]


def _row_spec(d):
    return pl.BlockSpec((_BN, d), lambda i: (i, 0))


def _full_spec(shape):
    nd = len(shape)
    return pl.BlockSpec(shape, lambda i, _n=nd: (0,) * _n)


def _deg_spec():
    return pl.BlockSpec((2, _BN, 8), lambda i: (0, i, 0))


def _matmul_plain(x, w):
    d_in, d_out = w.shape
    return pl.pallas_call(
        _plain_body,
        grid=(_G,),
        in_specs=[_row_spec(d_in), _full_spec((d_in, d_out))],
        out_specs=_row_spec(d_out),
        out_shape=jax.ShapeDtypeStruct((_NPAD, d_out), jnp.float32),
    )(x, w)


def _scale(z, degp):
    d = z.shape[1]
    return pl.pallas_call(
        _scale_body,
        grid=(_G,),
        in_specs=[_row_spec(d), _deg_spec()],
        out_specs=_row_spec(d),
        out_shape=jax.ShapeDtypeStruct((_NPAD, d), jnp.float32),
    )(z, degp)


def _fused_stats_mid(accp, hs, degp, b, g, be, w):
    d_in, d_out = w.shape
    dcol = accp.shape[2]
    return pl.pallas_call(
        _fused_body,
        grid=(2 * _G,),
        in_specs=[pl.BlockSpec((2, _BN, dcol), lambda i: (0, i % _G, 0)),
                  pl.BlockSpec((_BN, d_in), lambda i: (i % _G, 0)),
                  pl.BlockSpec((2, _BN, 8), lambda i: (0, i % _G, 0)),
                  _full_spec((1, d_in)), _full_spec((1, d_in)),
                  _full_spec((1, d_in)), _full_spec((d_in, d_out))],
        out_specs=pl.BlockSpec((_BN, d_out), lambda i: (i % _G, 0)),
        out_shape=jax.ShapeDtypeStruct((_NPAD, d_out), jnp.float32),
        scratch_shapes=[pltpu.VMEM((_N, d_in), jnp.float32),
                        pltpu.VMEM((2, d_in), jnp.float32),
                        pltpu.VMEM((2, d_in), jnp.float32)],
    )(accp, hs, degp, b, g, be, w)


def _final(accp, hs, degp, b):
    d = hs.shape[1]
    dcol = accp.shape[2]
    return pl.pallas_call(
        _final_body,
        grid=(_G,),
        in_specs=[pl.BlockSpec((2, _BN, dcol), lambda i: (0, i, 0)),
                  _row_spec(d), _deg_spec(), _full_spec((1, d))],
        out_specs=pl.BlockSpec((_BN, 2), lambda i: (i, 0)),
        out_shape=jax.ShapeDtypeStruct((_N, 2), jnp.float32),
    )(accp, hs, degp, b)


def kernel(x, edge_index, params):
    e = edge_index.shape[1]
    t = -(-e // (_NW * _CH))
    t = (t + 7) // 8 * 8
    epad = _NW * _CH * t
    t2 = 2 * t
    src = jnp.concatenate(
        [edge_index[0], jnp.zeros((epad - e,), jnp.int32)]).reshape(
            _NW * t, _CH)
    dst = jnp.concatenate(
        [edge_index[1], jnp.full((epad - e,), _N, jnp.int32)]).reshape(
            _NW * t, _CH)

    deg8 = _make_degree(8, t)
    prop_c64 = _make_propagate_cols(64, t2)
    prop_c32 = _make_propagate_cols(32, t2)
    z64 = jnp.zeros((_RPT, 64), jnp.float32)
    z32 = jnp.zeros((_RPT, 32), jnp.float32)
    z8 = jnp.zeros((_RPT, 8), jnp.float32)

    xpad = jnp.concatenate(
        [x, jnp.zeros((_NPAD - _N, x.shape[1]), jnp.float32)])

    w5 = jnp.concatenate(
        [params['W5'], jnp.zeros((params['W5'].shape[0], 62), jnp.float32)],
        axis=1)
    b5 = jnp.concatenate([params['b5'], jnp.zeros((62,), jnp.float32)])

    ws = [params['W1'], params['W2'], params['W3'], params['W4'], w5]
    bs = [params['b1'], params['b2'], params['b3'], params['b4'], b5]
    props = [prop_c64, prop_c64, prop_c64, prop_c32, prop_c32]
    zs = [z64, z64, z64, z32, z32]

    degp = deg8(jnp.ones((_CH, 8), jnp.float32), dst, z8)
    z1 = _matmul_plain(xpad, ws[0])
    hs = _scale(z1, degp)
    for i in range(5):
        accp = props[i](hs, src, dst, zs[i])
        bvec = bs[i].reshape(1, -1)
        if i < 4:
            hs = _fused_stats_mid(accp, hs, degp, bvec,
                                  params[f'g{i + 1}'].reshape(1, -1),
                                  params[f'be{i + 1}'].reshape(1, -1),
                                  ws[i + 1])
        else:
            out = _final(accp, hs, degp, bvec)
    return out


# speedup vs baseline: 19.7784x; 1.0769x over previous
import functools

import jax
import jax.numpy as jnp
from jax import lax
from jax.experimental import pallas as pl
from jax.experimental.pallas import tpu as pltpu
from jax.experimental.pallas import tpu_sc as plsc

_N = 10000
_NC, _NS = 2, 16
_NW = _NC * _NS
_CH = 128
_RPT = 632
_NPAD = _NS * _RPT
_BN = 2000
_G = _N // _BN
_EPS = 1e-5
_P = 40


@functools.lru_cache(maxsize=None)
def _make_degree(d: int, t: int):
    mesh = plsc.VectorSubcoreMesh(core_axis_name="c", subcore_axis_name="s")

    @functools.partial(
        pl.kernel,
        out_type=jax.ShapeDtypeStruct((_NC, _NPAD, d), jnp.float32),
        mesh=mesh,
        scratch_types=[
            pltpu.VMEM((t, _CH), jnp.int32),
            pltpu.VMEM((_CH, d), jnp.float32),
            pltpu.VMEM_SHARED((_NPAD, d), jnp.float32),
            pltpu.SemaphoreType.DMA,
            pltpu.SemaphoreType.DMA,
        ],
        compiler_params=pltpu.CompilerParams(use_tc_tiling_on_sc=False),
    )
    def deg(ones_hbm, dst_hbm, zrows_hbm, out_hbm,
            dst_v, ones_v, acc_sh, ssem0, ssem1):
        c = lax.axis_index("c")
        s = lax.axis_index("s")
        wid = c * _NS + s
        rs = pl.ds(s * _RPT, _RPT)

        pltpu.sync_copy(ones_hbm, ones_v)
        pltpu.sync_copy(dst_hbm.at[pl.ds(wid * t, t)], dst_v)
        pltpu.sync_copy(zrows_hbm, acc_sh.at[rs])
        plsc.subcore_barrier()

        ssems = (ssem0, ssem1)
        scat = [None, None]
        for j in range(t):
            b = j % 2
            if scat[b] is not None:
                scat[b].wait()
            scat[b] = pltpu.async_copy(
                ones_v, acc_sh.at[dst_v.at[j]], ssems[b], add=True)
        for b in range(2):
            if scat[b] is not None:
                scat[b].wait()
        plsc.subcore_barrier()

        pltpu.sync_copy(acc_sh.at[rs], out_hbm.at[c, rs])

    return deg


@functools.lru_cache(maxsize=None)
def _make_propagate_cols(dcol: int, t2: int):
    mesh = plsc.VectorSubcoreMesh(core_axis_name="c", subcore_axis_name="s")

    @functools.partial(
        pl.kernel,
        out_type=jax.ShapeDtypeStruct((_NPAD, 2 * dcol), jnp.float32),
        mesh=mesh,
        scratch_types=[
            pltpu.VMEM((_P, _CH), jnp.int32),
            pltpu.VMEM((_P, _CH), jnp.int32),
            pltpu.VMEM((2, _CH, dcol), jnp.float32),
            pltpu.VMEM_SHARED((_NPAD, dcol), jnp.float32),
            pltpu.VMEM_SHARED((_NPAD, dcol), jnp.float32),
            pltpu.SemaphoreType.DMA,
            pltpu.SemaphoreType.DMA,
            pltpu.SemaphoreType.DMA,
            pltpu.SemaphoreType.DMA,
        ],
        compiler_params=pltpu.CompilerParams(use_tc_tiling_on_sc=False),
    )
    def prop(table_hbm, src_hbm, dst_hbm, zrows_hbm, out_hbm,
             src_v, dst_v, rows_v, tbl_sh, acc_sh,
             gsem0, gsem1, ssem0, ssem1):
        c = lax.axis_index("c")
        s = lax.axis_index("s")
        rs = pl.ds(s * _RPT, _RPT)

        pltpu.sync_copy(table_hbm.at[rs, pl.ds(c * dcol, dcol)],
                        tbl_sh.at[rs])
        pltpu.sync_copy(zrows_hbm, acc_sh.at[rs])
        plsc.subcore_barrier()

        gsems = (gsem0, gsem1)
        ssems = (ssem0, ssem1)
        for piece in range(t2 // _P):
            base = s * t2 + piece * _P
            pltpu.sync_copy(src_hbm.at[pl.ds(base, _P)], src_v)
            pltpu.sync_copy(dst_hbm.at[pl.ds(base, _P)], dst_v)
            pltpu.async_copy(tbl_sh.at[src_v.at[0]], rows_v.at[0], gsem0)
            scat = [None, None]
            for j in range(_P):
                b = j % 2
                nb = (j + 1) % 2
                if j + 1 < _P:
                    if scat[nb] is not None:
                        scat[nb].wait()
                        scat[nb] = None
                    pltpu.async_copy(tbl_sh.at[src_v.at[j + 1]],
                                     rows_v.at[nb], gsems[nb])
                pltpu.make_async_copy(tbl_sh.at[src_v.at[j]],
                                      rows_v.at[b], gsems[b]).wait()
                scat[b] = pltpu.async_copy(
                    rows_v.at[b], acc_sh.at[dst_v.at[j]], ssems[b],
                    add=True)
            for b in range(2):
                if scat[b] is not None:
                    scat[b].wait()
        plsc.subcore_barrier()

        pltpu.sync_copy(acc_sh.at[rs], out_hbm.at[rs, pl.ds(c * dcol, dcol)])

    return prop


def _dinv_block(deg_ref):
    deg = deg_ref[0, :, 0:1] + deg_ref[1, :, 0:1] + 1.0
    return lax.rsqrt(deg)


def _plain_body(x_ref, w_ref, o_ref):
    o_ref[...] = jnp.dot(x_ref[...], w_ref[...],
                         preferred_element_type=jnp.float32)


def _scale_body(z_ref, deg_ref, o_ref):
    o_ref[...] = z_ref[...] * _dinv_block(deg_ref)


def _fused_body(acc_ref, hs_ref, deg_ref, b_ref, g_ref, be_ref, w_ref,
                o_ref, rawbuf, sums, stats):
    i = pl.program_id(0)

    @pl.when(i < _G)
    def _():
        raw = (acc_ref[...] + hs_ref[...]) * _dinv_block(deg_ref) \
            + b_ref[...]
        rawbuf[pl.ds(i * _BN, _BN), :] = raw
        part = jnp.stack([jnp.sum(raw, axis=0), jnp.sum(raw * raw, axis=0)])

        @pl.when(i == 0)
        def _():
            sums[...] = jnp.zeros_like(sums)

        sums[...] += part

        @pl.when(i == _G - 1)
        def _():
            mu = sums[0:1, :] / _N
            var = sums[1:2, :] / _N - mu * mu
            stats[...] = jnp.concatenate([mu, lax.rsqrt(var + _EPS)],
                                         axis=0)

    @pl.when(i >= _G)
    def _():
        raw = rawbuf[pl.ds((i - _G) * _BN, _BN), :]
        h = (raw - stats[0:1, :]) * stats[1:2, :] * g_ref[...] + be_ref[...]
        h = jnp.maximum(h, 0.0)
        o = jnp.dot(h, w_ref[...], preferred_element_type=jnp.float32)
        o_ref[...] = o * _dinv_block(deg_ref)


def _final_body(acc_ref, hs_ref, deg_ref, b_ref, o_ref):
    raw = (acc_ref[...] + hs_ref[...]) * _dinv_block(deg_ref) + b_ref[...]
    o_ref[...] = raw[:, 0:2]


def _row_spec(d):
    return pl.BlockSpec((_BN, d), lambda i: (i, 0))


def _full_spec(shape):
    nd = len(shape)
    return pl.BlockSpec(shape, lambda i, _n=nd: (0,) * _n)


def _deg_spec():
    return pl.BlockSpec((2, _BN, 8), lambda i: (0, i, 0))


def _matmul_plain(x, w):
    d_in, d_out = w.shape
    return pl.pallas_call(
        _plain_body,
        grid=(_G,),
        in_specs=[_row_spec(d_in), _full_spec((d_in, d_out))],
        out_specs=_row_spec(d_out),
        out_shape=jax.ShapeDtypeStruct((_NPAD, d_out), jnp.float32),
    )(x, w)


def _scale(z, degp):
    d = z.shape[1]
    return pl.pallas_call(
        _scale_body,
        grid=(_G,),
        in_specs=[_row_spec(d), _deg_spec()],
        out_specs=_row_spec(d),
        out_shape=jax.ShapeDtypeStruct((_NPAD, d), jnp.float32),
    )(z, degp)


def _fused_stats_mid(accp, hs, degp, b, g, be, w):
    d_in, d_out = w.shape
    return pl.pallas_call(
        _fused_body,
        grid=(2 * _G,),
        in_specs=[pl.BlockSpec((_BN, d_in), lambda i: (i % _G, 0)),
                  pl.BlockSpec((_BN, d_in), lambda i: (i % _G, 0)),
                  pl.BlockSpec((2, _BN, 8), lambda i: (0, i % _G, 0)),
                  _full_spec((1, d_in)), _full_spec((1, d_in)),
                  _full_spec((1, d_in)), _full_spec((d_in, d_out))],
        out_specs=pl.BlockSpec((_BN, d_out), lambda i: (i % _G, 0)),
        out_shape=jax.ShapeDtypeStruct((_NPAD, d_out), jnp.float32),
        scratch_shapes=[pltpu.VMEM((_N, d_in), jnp.float32),
                        pltpu.VMEM((2, d_in), jnp.float32),
                        pltpu.VMEM((2, d_in), jnp.float32)],
    )(accp, hs, degp, b, g, be, w)


def _final(accp, hs, degp, b):
    d = hs.shape[1]
    return pl.pallas_call(
        _final_body,
        grid=(_G,),
        in_specs=[_row_spec(d), _row_spec(d), _deg_spec(),
                  _full_spec((1, d))],
        out_specs=pl.BlockSpec((_BN, 2), lambda i: (i, 0)),
        out_shape=jax.ShapeDtypeStruct((_N, 2), jnp.float32),
    )(accp, hs, degp, b)


def kernel(x, edge_index, params):
    e = edge_index.shape[1]
    t = -(-e // (_NW * _CH))
    t = (t + 7) // 8 * 8
    epad = _NW * _CH * t
    t2 = 2 * t
    src = jnp.concatenate(
        [edge_index[0], jnp.zeros((epad - e,), jnp.int32)]).reshape(
            _NW * t, _CH)
    dst = jnp.concatenate(
        [edge_index[1], jnp.full((epad - e,), _N, jnp.int32)]).reshape(
            _NW * t, _CH)

    deg8 = _make_degree(8, t)
    prop_c64 = _make_propagate_cols(64, t2)
    prop_c32 = _make_propagate_cols(32, t2)
    z64 = jnp.zeros((_RPT, 64), jnp.float32)
    z32 = jnp.zeros((_RPT, 32), jnp.float32)
    z8 = jnp.zeros((_RPT, 8), jnp.float32)

    xpad = jnp.concatenate(
        [x, jnp.zeros((_NPAD - _N, x.shape[1]), jnp.float32)])

    w5 = jnp.concatenate(
        [params['W5'], jnp.zeros((params['W5'].shape[0], 62), jnp.float32)],
        axis=1)
    b5 = jnp.concatenate([params['b5'], jnp.zeros((62,), jnp.float32)])

    ws = [params['W1'], params['W2'], params['W3'], params['W4'], w5]
    bs = [params['b1'], params['b2'], params['b3'], params['b4'], b5]
    props = [prop_c64, prop_c64, prop_c64, prop_c32, prop_c32]
    zs = [z64, z64, z64, z32, z32]

    degp = deg8(jnp.ones((_CH, 8), jnp.float32), dst, z8)
    z1 = _matmul_plain(xpad, ws[0])
    hs = _scale(z1, degp)
    for i in range(5):
        accp = props[i](hs, src, dst, zs[i])
        bvec = bs[i].reshape(1, -1)
        if i < 4:
            hs = _fused_stats_mid(accp, hs, degp, bvec,
                                  params[f'g{i + 1}'].reshape(1, -1),
                                  params[f'be{i + 1}'].reshape(1, -1),
                                  ws[i + 1])
        else:
            out = _final(accp, hs, degp, bvec)
    return out


# speedup vs baseline: 20.3118x; 1.0270x over previous
import functools

import jax
import jax.numpy as jnp
from jax import lax
from jax.experimental import pallas as pl
from jax.experimental.pallas import tpu as pltpu
from jax.experimental.pallas import tpu_sc as plsc

_N = 10000
_NC, _NS = 2, 16
_NW = _NC * _NS
_CH = 128
_RPT = 632
_NPAD = _NS * _RPT
_BN = 2000
_G = _N // _BN
_EPS = 1e-5
_P = 80


@functools.lru_cache(maxsize=None)
def _make_degree(d: int, t: int):
    mesh = plsc.VectorSubcoreMesh(core_axis_name="c", subcore_axis_name="s")

    @functools.partial(
        pl.kernel,
        out_type=jax.ShapeDtypeStruct((_NC, _NPAD, d), jnp.float32),
        mesh=mesh,
        scratch_types=[
            pltpu.VMEM((t, _CH), jnp.int32),
            pltpu.VMEM((_CH, d), jnp.float32),
            pltpu.VMEM_SHARED((_NPAD, d), jnp.float32),
            pltpu.SemaphoreType.DMA,
            pltpu.SemaphoreType.DMA,
        ],
        compiler_params=pltpu.CompilerParams(use_tc_tiling_on_sc=False),
    )
    def deg(ones_hbm, dst_hbm, zrows_hbm, out_hbm,
            dst_v, ones_v, acc_sh, ssem0, ssem1):
        c = lax.axis_index("c")
        s = lax.axis_index("s")
        wid = c * _NS + s
        rs = pl.ds(s * _RPT, _RPT)

        pltpu.sync_copy(ones_hbm, ones_v)
        pltpu.sync_copy(dst_hbm.at[pl.ds(wid * t, t)], dst_v)
        pltpu.sync_copy(zrows_hbm, acc_sh.at[rs])
        plsc.subcore_barrier()

        ssems = (ssem0, ssem1)
        scat = [None, None]
        for j in range(t):
            b = j % 2
            if scat[b] is not None:
                scat[b].wait()
            scat[b] = pltpu.async_copy(
                ones_v, acc_sh.at[dst_v.at[j]], ssems[b], add=True)
        for b in range(2):
            if scat[b] is not None:
                scat[b].wait()
        plsc.subcore_barrier()

        pltpu.sync_copy(acc_sh.at[rs], out_hbm.at[c, rs])

    return deg


@functools.lru_cache(maxsize=None)
def _make_propagate_cols(dcol: int, t2: int):
    mesh = plsc.VectorSubcoreMesh(core_axis_name="c", subcore_axis_name="s")

    @functools.partial(
        pl.kernel,
        out_type=jax.ShapeDtypeStruct((_NPAD, 2 * dcol), jnp.float32),
        mesh=mesh,
        scratch_types=[
            pltpu.VMEM((_P, _CH), jnp.int32),
            pltpu.VMEM((_P, _CH), jnp.int32),
            pltpu.VMEM((2, _CH, dcol), jnp.float32),
            pltpu.VMEM_SHARED((_NPAD, dcol), jnp.float32),
            pltpu.VMEM_SHARED((_NPAD, dcol), jnp.float32),
            pltpu.SemaphoreType.DMA,
            pltpu.SemaphoreType.DMA,
            pltpu.SemaphoreType.DMA,
            pltpu.SemaphoreType.DMA,
        ],
        compiler_params=pltpu.CompilerParams(use_tc_tiling_on_sc=False),
    )
    def prop(table_hbm, src_hbm, dst_hbm, zrows_hbm, out_hbm,
             src_v, dst_v, rows_v, tbl_sh, acc_sh,
             gsem0, gsem1, ssem0, ssem1):
        c = lax.axis_index("c")
        s = lax.axis_index("s")
        rs = pl.ds(s * _RPT, _RPT)

        pltpu.sync_copy(table_hbm.at[rs, pl.ds(c * dcol, dcol)],
                        tbl_sh.at[rs])
        pltpu.sync_copy(zrows_hbm, acc_sh.at[rs])
        plsc.subcore_barrier()

        gsems = (gsem0, gsem1)
        ssems = (ssem0, ssem1)
        for piece in range(t2 // _P):
            base = s * t2 + piece * _P
            pltpu.sync_copy(src_hbm.at[pl.ds(base, _P)], src_v)
            pltpu.sync_copy(dst_hbm.at[pl.ds(base, _P)], dst_v)
            pltpu.async_copy(tbl_sh.at[src_v.at[0]], rows_v.at[0], gsem0)
            scat = [None, None]
            for j in range(_P):
                b = j % 2
                nb = (j + 1) % 2
                if j + 1 < _P:
                    if scat[nb] is not None:
                        scat[nb].wait()
                        scat[nb] = None
                    pltpu.async_copy(tbl_sh.at[src_v.at[j + 1]],
                                     rows_v.at[nb], gsems[nb])
                pltpu.make_async_copy(tbl_sh.at[src_v.at[j]],
                                      rows_v.at[b], gsems[b]).wait()
                scat[b] = pltpu.async_copy(
                    rows_v.at[b], acc_sh.at[dst_v.at[j]], ssems[b],
                    add=True)
            for b in range(2):
                if scat[b] is not None:
                    scat[b].wait()
        plsc.subcore_barrier()

        pltpu.sync_copy(acc_sh.at[rs], out_hbm.at[rs, pl.ds(c * dcol, dcol)])

    return prop


def _dinv_block(deg_ref):
    deg = deg_ref[0, :, 0:1] + deg_ref[1, :, 0:1] + 1.0
    return lax.rsqrt(deg)


def _plain_body(x_ref, w_ref, o_ref):
    o_ref[...] = jnp.dot(x_ref[...], w_ref[...],
                         preferred_element_type=jnp.float32)


def _scale_body(z_ref, deg_ref, o_ref):
    o_ref[...] = z_ref[...] * _dinv_block(deg_ref)


def _fused_body(acc_ref, hs_ref, deg_ref, b_ref, g_ref, be_ref, w_ref,
                o_ref, rawbuf, sums, stats):
    i = pl.program_id(0)

    @pl.when(i < _G)
    def _():
        raw = (acc_ref[...] + hs_ref[...]) * _dinv_block(deg_ref) \
            + b_ref[...]
        rawbuf[pl.ds(i * _BN, _BN), :] = raw
        part = jnp.stack([jnp.sum(raw, axis=0), jnp.sum(raw * raw, axis=0)])

        @pl.when(i == 0)
        def _():
            sums[...] = jnp.zeros_like(sums)

        sums[...] += part

        @pl.when(i == _G - 1)
        def _():
            mu = sums[0:1, :] / _N
            var = sums[1:2, :] / _N - mu * mu
            stats[...] = jnp.concatenate([mu, lax.rsqrt(var + _EPS)],
                                         axis=0)

    @pl.when(i >= _G)
    def _():
        raw = rawbuf[pl.ds((i - _G) * _BN, _BN), :]
        h = (raw - stats[0:1, :]) * stats[1:2, :] * g_ref[...] + be_ref[...]
        h = jnp.maximum(h, 0.0)
        o = jnp.dot(h, w_ref[...], preferred_element_type=jnp.float32)
        o_ref[...] = o * _dinv_block(deg_ref)


def _final_body(acc_ref, hs_ref, deg_ref, b_ref, o_ref):
    raw = (acc_ref[...] + hs_ref[...]) * _dinv_block(deg_ref) + b_ref[...]
    o_ref[...] = raw[:, 0:2]


def _row_spec(d):
    return pl.BlockSpec((_BN, d), lambda i: (i, 0))


def _full_spec(shape):
    nd = len(shape)
    return pl.BlockSpec(shape, lambda i, _n=nd: (0,) * _n)


def _deg_spec():
    return pl.BlockSpec((2, _BN, 8), lambda i: (0, i, 0))


def _matmul_plain(x, w):
    d_in, d_out = w.shape
    return pl.pallas_call(
        _plain_body,
        grid=(_G,),
        in_specs=[_row_spec(d_in), _full_spec((d_in, d_out))],
        out_specs=_row_spec(d_out),
        out_shape=jax.ShapeDtypeStruct((_NPAD, d_out), jnp.float32),
    )(x, w)


def _scale(z, degp):
    d = z.shape[1]
    return pl.pallas_call(
        _scale_body,
        grid=(_G,),
        in_specs=[_row_spec(d), _deg_spec()],
        out_specs=_row_spec(d),
        out_shape=jax.ShapeDtypeStruct((_NPAD, d), jnp.float32),
    )(z, degp)


def _fused_stats_mid(accp, hs, degp, b, g, be, w):
    d_in, d_out = w.shape
    return pl.pallas_call(
        _fused_body,
        grid=(2 * _G,),
        in_specs=[pl.BlockSpec((_BN, d_in), lambda i: (i % _G, 0)),
                  pl.BlockSpec((_BN, d_in), lambda i: (i % _G, 0)),
                  pl.BlockSpec((2, _BN, 8), lambda i: (0, i % _G, 0)),
                  _full_spec((1, d_in)), _full_spec((1, d_in)),
                  _full_spec((1, d_in)), _full_spec((d_in, d_out))],
        out_specs=pl.BlockSpec((_BN, d_out), lambda i: (i % _G, 0)),
        out_shape=jax.ShapeDtypeStruct((_NPAD, d_out), jnp.float32),
        scratch_shapes=[pltpu.VMEM((_N, d_in), jnp.float32),
                        pltpu.VMEM((2, d_in), jnp.float32),
                        pltpu.VMEM((2, d_in), jnp.float32)],
    )(accp, hs, degp, b, g, be, w)


def _final(accp, hs, degp, b):
    d = hs.shape[1]
    return pl.pallas_call(
        _final_body,
        grid=(_G,),
        in_specs=[_row_spec(d), _row_spec(d), _deg_spec(),
                  _full_spec((1, d))],
        out_specs=pl.BlockSpec((_BN, 2), lambda i: (i, 0)),
        out_shape=jax.ShapeDtypeStruct((_N, 2), jnp.float32),
    )(accp, hs, degp, b)


def kernel(x, edge_index, params):
    e = edge_index.shape[1]
    t = -(-e // (_NW * _CH))
    t = (t + 7) // 8 * 8
    epad = _NW * _CH * t
    t2 = 2 * t
    src = jnp.concatenate(
        [edge_index[0], jnp.zeros((epad - e,), jnp.int32)]).reshape(
            _NW * t, _CH)
    dst = jnp.concatenate(
        [edge_index[1], jnp.full((epad - e,), _N, jnp.int32)]).reshape(
            _NW * t, _CH)

    deg8 = _make_degree(8, t)
    prop_c64 = _make_propagate_cols(64, t2)
    prop_c32 = _make_propagate_cols(32, t2)
    z64 = jnp.zeros((_RPT, 64), jnp.float32)
    z32 = jnp.zeros((_RPT, 32), jnp.float32)
    z8 = jnp.zeros((_RPT, 8), jnp.float32)

    xpad = jnp.concatenate(
        [x, jnp.zeros((_NPAD - _N, x.shape[1]), jnp.float32)])

    w5 = jnp.concatenate(
        [params['W5'], jnp.zeros((params['W5'].shape[0], 62), jnp.float32)],
        axis=1)
    b5 = jnp.concatenate([params['b5'], jnp.zeros((62,), jnp.float32)])

    ws = [params['W1'], params['W2'], params['W3'], params['W4'], w5]
    bs = [params['b1'], params['b2'], params['b3'], params['b4'], b5]
    props = [prop_c64, prop_c64, prop_c64, prop_c32, prop_c32]
    zs = [z64, z64, z64, z32, z32]

    degp = deg8(jnp.ones((_CH, 8), jnp.float32), dst, z8)
    z1 = _matmul_plain(xpad, ws[0])
    hs = _scale(z1, degp)
    for i in range(5):
        accp = props[i](hs, src, dst, zs[i])
        bvec = bs[i].reshape(1, -1)
        if i < 4:
            hs = _fused_stats_mid(accp, hs, degp, bvec,
                                  params[f'g{i + 1}'].reshape(1, -1),
                                  params[f'be{i + 1}'].reshape(1, -1),
                                  ws[i + 1])
        else:
            out = _final(accp, hs, degp, bvec)
    return out
